# baseline (speedup 1.0000x reference)
import jax
import jax.numpy as jnp
from jax.experimental import pallas as pl
from jax.experimental.pallas import tpu as pltpu

TOKENS = 32
VOCAB = 1000000
EPS = 1e-10
CHUNK = 65536
GRID = (VOCAB + CHUNK - 1) // CHUNK
SUB = 8192
NSUB = CHUNK // SUB
LAST_SID = (VOCAB - 1) // SUB
MAXOFF = (VOCAB - SUB) // 128 * 128


def _phase1(x_ref, e_ref, t_ref, c_ref, a_ref, m_ref):
    i = pl.program_id(0)

    @pl.when(i == 0)
    def _init():
        m_ref[...] = jnp.full((TOKENS, 1), -jnp.inf, jnp.float32)
        c_ref[...] = jnp.zeros((TOKENS, 1), jnp.int32)
        a_ref[...] = jnp.zeros((TOKENS, 1), jnp.int32)

    x = x_ref[...]
    e = e_ref[...]
    t = t_ref[...]

    noise = jnp.log(e + EPS)
    key = x - t * noise

    idx = jax.lax.broadcasted_iota(jnp.int32, key.shape, 1)
    key = jnp.where(idx < VOCAB - i * CHUNK, key, -jnp.inf)

    sub_max = jnp.max(key.reshape(TOKENS, NSUB, SUB), axis=2)
    loc_max = jnp.max(sub_max, axis=1, keepdims=True)
    sidx = jax.lax.broadcasted_iota(jnp.int32, sub_max.shape, 1)
    loc_sub = jnp.min(
        jnp.where(sub_max == loc_max, sidx, NSUB), axis=1, keepdims=True
    ) + i * NSUB

    @pl.when(i == GRID - 1)
    def _tail_exact():
        hit = key == loc_max
        a_ref[...] = jnp.min(
            jnp.where(hit, idx, CHUNK), axis=1, keepdims=True
        ) + i * CHUNK

    better = loc_max > m_ref[...]
    m_ref[...] = jnp.where(better, loc_max, m_ref[...])
    c_ref[...] = jnp.where(better, loc_sub, c_ref[...])


def _phase2(sid_ref, x_hbm, e_hbm, off_ref, t_ref, sidv_ref, tailarg_ref,
            o_ref, xs, es, sem):
    copies = []
    for r in range(TOKENS):
        off = jnp.minimum(sid_ref[r] * SUB, MAXOFF)
        g = r // 8
        copies.append(pltpu.make_async_copy(
            x_hbm.at[pl.ds(8 * g, 8), pl.ds(off, SUB)], xs.at[r], sem
        ))
        copies.append(pltpu.make_async_copy(
            e_hbm.at[pl.ds(off, SUB)], es.at[pl.ds(r * SUB, SUB)], sem
        ))
    for c in copies:
        c.start()
    for c in copies:
        c.wait()

    rows = [xs[r, r % 8:r % 8 + 1, :] for r in range(TOKENS)]
    x = jnp.concatenate(rows, axis=0)
    e = es[...].reshape(TOKENS, SUB)
    t = t_ref[...]
    offs = off_ref[...]

    key = x - t * jnp.log(e + EPS)
    idx = jax.lax.broadcasted_iota(jnp.int32, key.shape, 1)
    loc_max = jnp.max(key, axis=1, keepdims=True)
    arg = jnp.min(jnp.where(key == loc_max, idx, SUB), axis=1, keepdims=True)
    o_ref[...] = jnp.where(
        sidv_ref[...] == LAST_SID, tailarg_ref[...], arg + offs
    )


@jax.jit
def kernel(logits, temperatures, exponential):
    t = temperatures[:, None].astype(jnp.float32)

    sub_ids, tail_arg = pl.pallas_call(
        _phase1,
        grid=(GRID,),
        in_specs=[
            pl.BlockSpec((TOKENS, CHUNK), lambda i: (0, i)),
            pl.BlockSpec((1, CHUNK), lambda i: (0, i)),
            pl.BlockSpec((TOKENS, 1), lambda i: (0, 0)),
        ],
        out_specs=[
            pl.BlockSpec((TOKENS, 1), lambda i: (0, 0)),
            pl.BlockSpec((TOKENS, 1), lambda i: (0, 0)),
        ],
        out_shape=[
            jax.ShapeDtypeStruct((TOKENS, 1), jnp.int32),
            jax.ShapeDtypeStruct((TOKENS, 1), jnp.int32),
        ],
        scratch_shapes=[pltpu.VMEM((TOKENS, 1), jnp.float32)],
    )(logits, exponential, t)

    offs = jnp.minimum(sub_ids * SUB, MAXOFF)

    grid_spec = pltpu.PrefetchScalarGridSpec(
        num_scalar_prefetch=1,
        grid=(1,),
        in_specs=[
            pl.BlockSpec(memory_space=pl.ANY),
            pl.BlockSpec(memory_space=pl.ANY),
            pl.BlockSpec((TOKENS, 1), lambda i, sid: (0, 0)),
            pl.BlockSpec((TOKENS, 1), lambda i, sid: (0, 0)),
            pl.BlockSpec((TOKENS, 1), lambda i, sid: (0, 0)),
            pl.BlockSpec((TOKENS, 1), lambda i, sid: (0, 0)),
        ],
        out_specs=pl.BlockSpec((TOKENS, 1), lambda i, sid: (0, 0)),
        scratch_shapes=[
            pltpu.VMEM((TOKENS, 8, SUB), jnp.float32),
            pltpu.VMEM((TOKENS * SUB,), jnp.float32),
            pltpu.SemaphoreType.DMA,
        ],
    )
    out = pl.pallas_call(
        _phase2,
        grid_spec=grid_spec,
        out_shape=jax.ShapeDtypeStruct((TOKENS, 1), jnp.int32),
    )(sub_ids[:, 0], logits, exponential.reshape(-1), offs, t, sub_ids, tail_arg)
    return out[:, 0]


# speedup vs baseline: 1.3331x; 1.3331x over previous
import jax
import jax.numpy as jnp
from jax.experimental import pallas as pl
from jax.experimental.pallas import tpu as pltpu

TOKENS = 32
VOCAB = 1000000
EPS = 1e-10
CHUNK = 65536
GRID = (VOCAB + CHUNK - 1) // CHUNK
SUB = 8192
NSUB = CHUNK // SUB
LAST_SID = (VOCAB - 1) // SUB
MAXOFF = (VOCAB - SUB) // 128 * 128


def _phase1(x_ref, e_ref, t_ref, c_ref, a_ref, m_ref):
    i = pl.program_id(0)

    @pl.when(i == 0)
    def _init():
        m_ref[...] = jnp.full((TOKENS, 1), -jnp.inf, jnp.float32)
        c_ref[...] = jnp.zeros((TOKENS, 1), jnp.int32)
        a_ref[...] = jnp.zeros((TOKENS, 1), jnp.int32)

    x = x_ref[...]
    e = e_ref[...]
    t = t_ref[...]

    noise = jnp.log(e + EPS)
    key = x - t * noise

    idx = jax.lax.broadcasted_iota(jnp.int32, key.shape, 1)
    key = jnp.where(idx < VOCAB - i * CHUNK, key, -jnp.inf)

    sub_max = jnp.concatenate(
        [jnp.max(key[:, s * SUB:(s + 1) * SUB], axis=1, keepdims=True)
         for s in range(NSUB)], axis=1)
    loc_max = jnp.max(sub_max, axis=1, keepdims=True)
    sidx = jax.lax.broadcasted_iota(jnp.int32, sub_max.shape, 1)
    loc_sub = jnp.min(
        jnp.where(sub_max == loc_max, sidx, NSUB), axis=1, keepdims=True
    ) + i * NSUB

    @pl.when(i == GRID - 1)
    def _tail_exact():
        hit = key == loc_max
        a_ref[...] = jnp.min(
            jnp.where(hit, idx, CHUNK), axis=1, keepdims=True
        ) + i * CHUNK

    better = loc_max > m_ref[...]
    m_ref[...] = jnp.where(better, loc_max, m_ref[...])
    c_ref[...] = jnp.where(better, loc_sub, c_ref[...])


def _phase2(sid_ref, x_hbm, e_hbm, off_ref, t_ref, sidv_ref, tailarg_ref,
            o_ref, xs, es, sem):
    copies = []
    for r in range(TOKENS):
        off = jnp.minimum(sid_ref[r] * SUB, MAXOFF)
        g = r // 8
        copies.append(pltpu.make_async_copy(
            x_hbm.at[pl.ds(8 * g, 8), pl.ds(off, SUB)], xs.at[r], sem
        ))
        copies.append(pltpu.make_async_copy(
            e_hbm.at[pl.ds(off, SUB)], es.at[pl.ds(r * SUB, SUB)], sem
        ))
    for c in copies:
        c.start()
    for c in copies:
        c.wait()

    rows = [xs[r, r % 8:r % 8 + 1, :] for r in range(TOKENS)]
    x = jnp.concatenate(rows, axis=0)
    e = es[...].reshape(TOKENS, SUB)
    t = t_ref[...]
    offs = off_ref[...]

    key = x - t * jnp.log(e + EPS)
    idx = jax.lax.broadcasted_iota(jnp.int32, key.shape, 1)
    loc_max = jnp.max(key, axis=1, keepdims=True)
    arg = jnp.min(jnp.where(key == loc_max, idx, SUB), axis=1, keepdims=True)
    o_ref[...] = jnp.where(
        sidv_ref[...] == LAST_SID, tailarg_ref[...], arg + offs
    )


@jax.jit
def kernel(logits, temperatures, exponential):
    t = temperatures[:, None].astype(jnp.float32)

    sub_ids, tail_arg = pl.pallas_call(
        _phase1,
        grid=(GRID,),
        in_specs=[
            pl.BlockSpec((TOKENS, CHUNK), lambda i: (0, i)),
            pl.BlockSpec((1, CHUNK), lambda i: (0, i)),
            pl.BlockSpec((TOKENS, 1), lambda i: (0, 0)),
        ],
        out_specs=[
            pl.BlockSpec((TOKENS, 1), lambda i: (0, 0)),
            pl.BlockSpec((TOKENS, 1), lambda i: (0, 0)),
        ],
        out_shape=[
            jax.ShapeDtypeStruct((TOKENS, 1), jnp.int32),
            jax.ShapeDtypeStruct((TOKENS, 1), jnp.int32),
        ],
        scratch_shapes=[pltpu.VMEM((TOKENS, 1), jnp.float32)],
    )(logits, exponential, t)

    offs = jnp.minimum(sub_ids * SUB, MAXOFF)

    grid_spec = pltpu.PrefetchScalarGridSpec(
        num_scalar_prefetch=1,
        grid=(1,),
        in_specs=[
            pl.BlockSpec(memory_space=pl.ANY),
            pl.BlockSpec(memory_space=pl.ANY),
            pl.BlockSpec((TOKENS, 1), lambda i, sid: (0, 0)),
            pl.BlockSpec((TOKENS, 1), lambda i, sid: (0, 0)),
            pl.BlockSpec((TOKENS, 1), lambda i, sid: (0, 0)),
            pl.BlockSpec((TOKENS, 1), lambda i, sid: (0, 0)),
        ],
        out_specs=pl.BlockSpec((TOKENS, 1), lambda i, sid: (0, 0)),
        scratch_shapes=[
            pltpu.VMEM((TOKENS, 8, SUB), jnp.float32),
            pltpu.VMEM((TOKENS * SUB,), jnp.float32),
            pltpu.SemaphoreType.DMA,
        ],
    )
    out = pl.pallas_call(
        _phase2,
        grid_spec=grid_spec,
        out_shape=jax.ShapeDtypeStruct((TOKENS, 1), jnp.int32),
    )(sub_ids[:, 0], logits, exponential.reshape(-1), offs, t, sub_ids, tail_arg)
    return out[:, 0]


# speedup vs baseline: 1.7459x; 1.3097x over previous
import jax
import jax.numpy as jnp
from jax import lax
from jax.experimental import pallas as pl
from jax.experimental.pallas import tpu as pltpu
from jax.experimental.pallas import tpu_sc as plsc

TOKENS = 32
VOCAB = 1000000
EPS = 1e-10
CHUNK = 65536
TCGRID = 15
TCV = TCGRID * CHUNK
TAIL = VOCAB - TCV
NSTRIP = TAIL // 16


def _tc_kernel(x_ref, e_ref, t_ref, o_ref, mx_ref, m_ref):
    i = pl.program_id(0)

    @pl.when(i == 0)
    def _init():
        m_ref[...] = jnp.full((TOKENS, 1), -jnp.inf, jnp.float32)
        o_ref[...] = jnp.zeros((TOKENS, 1), jnp.int32)

    x = x_ref[...]
    e = e_ref[...]
    t = t_ref[...]

    noise = jnp.log(e + EPS)
    key = x - t * noise

    idx = jax.lax.broadcasted_iota(jnp.int32, key.shape, 1)
    loc_max = jnp.max(key, axis=1, keepdims=True)
    hit = key == loc_max
    loc_arg = jnp.min(jnp.where(hit, idx, CHUNK), axis=1, keepdims=True)
    loc_arg = loc_arg + i * CHUNK

    better = loc_max > m_ref[...]
    m_ref[...] = jnp.where(better, loc_max, m_ref[...])
    o_ref[...] = jnp.where(better, loc_arg, o_ref[...])
    mx_ref[...] = m_ref[...]


def _noise_kernel(e_ref, n_ref):
    n_ref[...] = jnp.log(e_ref[...] + EPS)


def _sc_tail(x_hbm, n_hbm, t_hbm, mx_hbm, ix_hbm, xv, nv, tv, mxv, ixv):
    c = lax.axis_index("c")
    s = lax.axis_index("s")
    w = c * 16 + s

    pltpu.sync_copy(x_hbm.at[w], xv)
    pltpu.sync_copy(n_hbm.at[0], nv)
    pltpu.sync_copy(t_hbm.at[w], tv)
    il = lax.broadcasted_iota(jnp.int32, (16,), 0)
    t = tv[pl.ds(0, 16)]

    def body(j, carry):
        m, bi = carry
        xk = xv[pl.ds(j * 16, 16)]
        nk = nv[pl.ds(j * 16, 16)]
        key = xk - t * nk
        upd = key > m
        m = jnp.where(upd, key, m)
        bi = jnp.where(upd, il * 0 + j, bi)
        return m, bi

    m0 = jnp.full((16,), -jnp.inf, jnp.float32)
    b0 = jnp.zeros((16,), jnp.int32)
    m, bi = lax.fori_loop(0, NSTRIP, body, (m0, b0))

    mxv[...] = m
    ixv[...] = bi * 16 + il + TCV
    pltpu.sync_copy(mxv, mx_hbm.at[w])
    pltpu.sync_copy(ixv, ix_hbm.at[w])


@jax.jit
def kernel(logits, temperatures, exponential):
    t = temperatures[:, None].astype(jnp.float32)

    x_tail = lax.slice(logits, (0, TCV), (TOKENS, VOCAB))
    e_tail = lax.slice(exponential, (0, TCV), (1, VOCAB))

    n_tail = pl.pallas_call(
        _noise_kernel,
        out_shape=jax.ShapeDtypeStruct((1, TAIL), jnp.float32),
    )(e_tail)

    sc = pl.kernel(
        _sc_tail,
        out_type=[
            jax.ShapeDtypeStruct((TOKENS, 16), jnp.float32),
            jax.ShapeDtypeStruct((TOKENS, 16), jnp.int32),
        ],
        scratch_types=[
            pltpu.VMEM((TAIL,), jnp.float32),
            pltpu.VMEM((TAIL,), jnp.float32),
            pltpu.VMEM((16,), jnp.float32),
            pltpu.VMEM((16,), jnp.float32),
            pltpu.VMEM((16,), jnp.int32),
        ],
        mesh=plsc.VectorSubcoreMesh(core_axis_name="c", subcore_axis_name="s"),
    )
    tb = jnp.broadcast_to(temperatures.astype(jnp.float32)[:, None], (TOKENS, 16))
    sc_max, sc_idx = sc(x_tail, n_tail, tb)

    tc_arg, tc_max = pl.pallas_call(
        _tc_kernel,
        grid=(TCGRID,),
        in_specs=[
            pl.BlockSpec((TOKENS, CHUNK), lambda i: (0, i)),
            pl.BlockSpec((1, CHUNK), lambda i: (0, i)),
            pl.BlockSpec((TOKENS, 1), lambda i: (0, 0)),
        ],
        out_specs=[
            pl.BlockSpec((TOKENS, 1), lambda i: (0, 0)),
            pl.BlockSpec((TOKENS, 1), lambda i: (0, 0)),
        ],
        out_shape=[
            jax.ShapeDtypeStruct((TOKENS, 1), jnp.int32),
            jax.ShapeDtypeStruct((TOKENS, 1), jnp.float32),
        ],
        scratch_shapes=[pltpu.VMEM((TOKENS, 1), jnp.float32)],
    )(logits, exponential, t)

    lane_best = jnp.max(sc_max, axis=1)
    lane_arg = jnp.min(
        jnp.where(sc_max == lane_best[:, None], sc_idx, VOCAB), axis=1)
    better = lane_best > tc_max[:, 0]
    return jnp.where(better, lane_arg, tc_arg[:, 0]).astype(jnp.int32)
